# baseline (speedup 1.0000x reference)
import jax
import jax.numpy as jnp
from jax.experimental import pallas as pl
from jax.experimental.pallas import tpu as pltpu

_SX = 16.0 / 255.0
_INV_SX = 255.0 / 16.0
_C = _SX * 1.4426950408889634


def _body(x_ref, o_ref):
    x = x_ref[...]
    m = jnp.max(x, axis=1, keepdims=True)
    q = jnp.round((x - m) * _INV_SX)
    e = jnp.exp2(q * _C)
    s = jnp.sum(e, axis=1, keepdims=True)
    r = 255.0 / s
    o_ref[...] = jnp.round(e * r) * (1.0 / 255.0)


def kernel(inputs):
    b, h, w, w2 = inputs.shape
    xt = jnp.transpose(inputs, (1, 2, 3, 0)).reshape(h * w, w2, b)
    blk = 8
    out = pl.pallas_call(
        _body,
        grid=(xt.shape[0] // blk,),
        in_specs=[pl.BlockSpec((blk, w2, b), lambda i: (i, 0, 0))],
        out_specs=pl.BlockSpec((blk, w2, b), lambda i: (i, 0, 0)),
        out_shape=jax.ShapeDtypeStruct(xt.shape, xt.dtype),
    )(xt)
    return jnp.transpose(out.reshape(h, w, w2, b), (3, 0, 1, 2))


# speedup vs baseline: 1.1994x; 1.1994x over previous
import jax
import jax.numpy as jnp
from jax.experimental import pallas as pl
from jax.experimental.pallas import tpu as pltpu

_SX = 16.0 / 255.0
_INV_SX = 255.0 / 16.0
_C = _SX * 1.4426950408889634


def _body(x_ref, o_ref):
    x = x_ref[...]
    m = jnp.max(x, axis=1, keepdims=True)
    q = jnp.round((x - m) * _INV_SX)
    e = jnp.exp2(q * _C)
    s = jnp.sum(e, axis=1, keepdims=True)
    r = 255.0 / s
    o_ref[...] = jnp.round(e * r) * (1.0 / 255.0)


def kernel(inputs):
    b, h, w, w2 = inputs.shape
    xt = jnp.transpose(inputs, (1, 2, 3, 0)).reshape(h * w, w2, b)
    blk = 16
    out = pl.pallas_call(
        _body,
        grid=(xt.shape[0] // blk,),
        in_specs=[pl.BlockSpec((blk, w2, b), lambda i: (i, 0, 0))],
        out_specs=pl.BlockSpec((blk, w2, b), lambda i: (i, 0, 0)),
        out_shape=jax.ShapeDtypeStruct(xt.shape, xt.dtype),
    )(xt)
    return jnp.transpose(out.reshape(h, w, w2, b), (3, 0, 1, 2))
